# baseline (speedup 1.0000x reference)
import functools

import jax
import jax.numpy as jnp
from jax import lax
from jax.experimental import pallas as pl
from jax.experimental.pallas import tpu as pltpu
from jax.experimental.pallas import tpu_sc as plsc

NC = 2
NS = 16
EB = 128


K = 8


def _make_segment_sum(n, d, nacc, nb):
    zps = nacc // NS
    nch = nb // K

    mesh = plsc.VectorSubcoreMesh(
        core_axis_name="c", subcore_axis_name="s",
        num_cores=NC, num_subcores=NS)

    @functools.partial(
        pl.kernel,
        out_type=jax.ShapeDtypeStruct((NC, nacc, d), jnp.float32),
        mesh=mesh,
        scratch_types=[
            pltpu.VMEM_SHARED((nacc, d), jnp.float32),
            pltpu.VMEM((3, K, EB), jnp.int32),
            pltpu.VMEM((3, K, EB), jnp.int32),
            pltpu.VMEM((2, EB, d), jnp.float32),
            pltpu.SemaphoreType.DMA,
            pltpu.SemaphoreType.DMA,
        ],
    )
    def seg_sum(h_hbm, src_hbm, dst_hbm, zero_hbm, out_hbm,
                acc, sbuf, dbuf, rbuf, gsem, asem):
        c = lax.axis_index("c")
        s = lax.axis_index("s")
        wid = c * NS + s
        row0 = wid * nb

        pltpu.sync_copy(zero_hbm, acc.at[pl.ds(s * zps, zps)])
        plsc.subcore_barrier()

        def load_idx(ch):
            sl = lax.rem(ch, 3)
            pltpu.sync_copy(src_hbm.at[pl.ds(row0 + ch * K, K)], sbuf.at[sl])
            pltpu.sync_copy(dst_hbm.at[pl.ds(row0 + ch * K, K)], dbuf.at[sl])

        def gather_desc(jj):
            ch = jj // K
            return pltpu.make_async_copy(
                h_hbm.at[sbuf.at[lax.rem(ch, 3)].at[lax.rem(jj, K)]],
                rbuf.at[lax.rem(jj, 2)], gsem)

        def scatter_desc(jj):
            ch = jj // K
            return pltpu.make_async_copy(
                rbuf.at[lax.rem(jj, 2)],
                acc.at[dbuf.at[lax.rem(ch, 3)].at[lax.rem(jj, K)]], asem)

        load_idx(0)
        load_idx(1)
        gather_desc(0).start()
        gather_desc(1).start()
        gather_desc(0).wait()
        scatter_desc(0).start(add=True)

        def body(jj, carry):
            @pl.when(jnp.logical_and(lax.rem(jj, K) == 0, jj + 2 * K <= nb))
            def _():
                load_idx(jj // K + 1)
            scatter_desc(jj - 2).wait()
            gather_desc(jj).start()
            gather_desc(jj - 1).wait()
            scatter_desc(jj - 1).start(add=True)
            return carry

        lax.fori_loop(2, nb, body, 0)
        gather_desc(nb - 1).wait()
        scatter_desc(nb - 1).start(add=True)
        scatter_desc(nb - 2).wait()
        scatter_desc(nb - 1).wait()
        plsc.subcore_barrier()

        pltpu.sync_copy(acc.at[pl.ds(s * zps, zps)],
                        out_hbm.at[c].at[pl.ds(s * zps, zps)])

    return seg_sum


def _linear_relu(parts, w, b, n, d, blk):
    nb = n // blk

    def body(p0_ref, p1_ref, w_ref, b_ref, o_ref):
        msgs = p0_ref[0] + p1_ref[0]
        y = lax.dot_general(msgs, w_ref[...], (((1,), (0,)), ((), ())),
                            preferred_element_type=jnp.float32)
        o_ref[...] = jnp.maximum(y + b_ref[...], 0.0)

    return pl.pallas_call(
        body,
        grid=(nb,),
        in_specs=[
            pl.BlockSpec((1, blk, d), lambda i: (0, i, 0)),
            pl.BlockSpec((1, blk, d), lambda i: (1, i, 0)),
            pl.BlockSpec((d, d), lambda i: (0, 0)),
            pl.BlockSpec((1, d), lambda i: (0, 0)),
        ],
        out_specs=pl.BlockSpec((blk, d), lambda i: (i, 0)),
        out_shape=jax.ShapeDtypeStruct((n, d), jnp.float32),
    )(parts, parts, w, b.reshape(1, d))


def kernel(x, edge_index, W1, b1, W2, b2):
    n, d = x.shape
    e = edge_index.shape[1]

    loop = jnp.arange(n, dtype=jnp.int32)
    src = jnp.concatenate([edge_index[0].astype(jnp.int32), loop])
    dst = jnp.concatenate([edge_index[1].astype(jnp.int32), loop])

    etot = e + n
    nw = NC * NS
    nb = -(-etot // (nw * EB * K)) * K
    epad = nw * nb * EB - etot
    src = jnp.concatenate([src, jnp.zeros((epad,), jnp.int32)])
    dst = jnp.concatenate([dst, jnp.full((epad,), n, jnp.int32)])
    src = src.reshape(nw * nb, EB)
    dst = dst.reshape(nw * nb, EB)

    nacc = -(-(n + 1) // (8 * NS)) * (8 * NS)
    zeros = jnp.zeros((nacc // NS, d), jnp.float32)

    seg = _make_segment_sum(n, d, nacc, nb)

    parts1 = seg(x, src, dst, zeros)
    h1 = _linear_relu(parts1, W1, b1, n, d, blk=1000)
    parts2 = seg(h1, src, dst, zeros)
    h2 = _linear_relu(parts2, W2, b2, n, d, blk=1000)
    return h2


# speedup vs baseline: 1.0141x; 1.0141x over previous
import functools

import jax
import jax.numpy as jnp
from jax import lax
from jax.experimental import pallas as pl
from jax.experimental.pallas import tpu as pltpu
from jax.experimental.pallas import tpu_sc as plsc

NC = 2
NS = 16
EB = 128


K = 8


def _make_segment_sum(n, d, nacc, nb):
    zps = nacc // NS
    nch = nb // K

    mesh = plsc.VectorSubcoreMesh(
        core_axis_name="c", subcore_axis_name="s",
        num_cores=NC, num_subcores=NS)

    @functools.partial(
        pl.kernel,
        out_type=jax.ShapeDtypeStruct((NC, nacc, d), jnp.float32),
        mesh=mesh,
        scratch_types=[
            pltpu.VMEM_SHARED((nacc, d), jnp.float32),
            pltpu.VMEM((3, K, EB), jnp.int32),
            pltpu.VMEM((3, K, EB), jnp.int32),
            pltpu.VMEM((2, EB, d), jnp.float32),
            pltpu.SemaphoreType.DMA,
            pltpu.SemaphoreType.DMA,
        ],
    )
    def seg_sum(h_hbm, src_hbm, dst_hbm, zero_hbm, out_hbm,
                acc, sbuf, dbuf, rbuf, gsem, asem):
        c = lax.axis_index("c")
        s = lax.axis_index("s")
        wid = c * NS + s
        row0 = wid * nb

        pltpu.sync_copy(zero_hbm, acc.at[pl.ds(s * zps, zps)])
        plsc.subcore_barrier()

        def load_idx(ch):
            sl = lax.rem(ch, 3)
            pltpu.sync_copy(src_hbm.at[pl.ds(row0 + ch * K, K)], sbuf.at[sl])
            pltpu.sync_copy(dst_hbm.at[pl.ds(row0 + ch * K, K)], dbuf.at[sl])

        def gather_desc(jj):
            ch = jj // K
            return pltpu.make_async_copy(
                h_hbm.at[sbuf.at[lax.rem(ch, 3)].at[lax.rem(jj, K)]],
                rbuf.at[lax.rem(jj, 2)], gsem)

        def scatter_sync(jj):
            ch = jj // K
            pltpu.sync_copy(
                rbuf.at[lax.rem(jj, 2)],
                acc.at[dbuf.at[lax.rem(ch, 3)].at[lax.rem(jj, K)]], add=True)

        load_idx(0)
        load_idx(1)
        gather_desc(0).start()

        def body(jj, carry):
            @pl.when(jnp.logical_and(lax.rem(jj, K) == 0, jj + 2 * K <= nb))
            def _():
                load_idx(jj // K + 1)

            @pl.when(jj + 1 < nb)
            def _():
                gather_desc(jj + 1).start()

            gather_desc(jj).wait()
            scatter_sync(jj)
            return carry

        lax.fori_loop(0, nb, body, 0)
        plsc.subcore_barrier()

        pltpu.sync_copy(acc.at[pl.ds(s * zps, zps)],
                        out_hbm.at[c].at[pl.ds(s * zps, zps)])

    return seg_sum


def _linear_relu(parts, w, b, n, d, blk):
    nb = n // blk

    def body(p0_ref, p1_ref, w_ref, b_ref, o_ref):
        msgs = p0_ref[0] + p1_ref[0]
        y = lax.dot_general(msgs, w_ref[...], (((1,), (0,)), ((), ())),
                            preferred_element_type=jnp.float32)
        o_ref[...] = jnp.maximum(y + b_ref[...], 0.0)

    return pl.pallas_call(
        body,
        grid=(nb,),
        in_specs=[
            pl.BlockSpec((1, blk, d), lambda i: (0, i, 0)),
            pl.BlockSpec((1, blk, d), lambda i: (1, i, 0)),
            pl.BlockSpec((d, d), lambda i: (0, 0)),
            pl.BlockSpec((1, d), lambda i: (0, 0)),
        ],
        out_specs=pl.BlockSpec((blk, d), lambda i: (i, 0)),
        out_shape=jax.ShapeDtypeStruct((n, d), jnp.float32),
    )(parts, parts, w, b.reshape(1, d))


def kernel(x, edge_index, W1, b1, W2, b2):
    n, d = x.shape
    e = edge_index.shape[1]

    loop = jnp.arange(n, dtype=jnp.int32)
    src = jnp.concatenate([edge_index[0].astype(jnp.int32), loop])
    dst = jnp.concatenate([edge_index[1].astype(jnp.int32), loop])

    etot = e + n
    nw = NC * NS
    nb = -(-etot // (nw * EB * K)) * K
    epad = nw * nb * EB - etot
    src = jnp.concatenate([src, jnp.zeros((epad,), jnp.int32)])
    dst = jnp.concatenate([dst, jnp.full((epad,), n, jnp.int32)])
    src = src.reshape(nw * nb, EB)
    dst = dst.reshape(nw * nb, EB)

    nacc = -(-(n + 1) // (8 * NS)) * (8 * NS)
    zeros = jnp.zeros((nacc // NS, d), jnp.float32)

    seg = _make_segment_sum(n, d, nacc, nb)

    parts1 = seg(x, src, dst, zeros)
    h1 = _linear_relu(parts1, W1, b1, n, d, blk=1000)
    parts2 = seg(h1, src, dst, zeros)
    h2 = _linear_relu(parts2, W2, b2, n, d, blk=1000)
    return h2


# speedup vs baseline: 3.9672x; 3.9119x over previous
import functools

import jax
import jax.numpy as jnp
from jax import lax
from jax.experimental import pallas as pl
from jax.experimental.pallas import tpu as pltpu
from jax.experimental.pallas import tpu_sc as plsc

NC = 2
NS = 16
EB = 128


def _make_segment_sum(n, d, nacc, nb, nc):
    zps = nacc // NS
    per_w = nb * EB

    mesh = plsc.VectorSubcoreMesh(
        core_axis_name="c", subcore_axis_name="s",
        num_cores=nc, num_subcores=NS)

    @functools.partial(
        pl.kernel,
        out_type=jax.ShapeDtypeStruct((nc, nacc, d), jnp.float32),
        mesh=mesh,
        scratch_types=[
            pltpu.VMEM_SHARED((nacc, d), jnp.float32),
            pltpu.VMEM((EB,), jnp.int32),
            pltpu.VMEM((EB,), jnp.int32),
            pltpu.VMEM((EB,), jnp.int32),
            pltpu.VMEM((EB,), jnp.int32),
            pltpu.VMEM((EB, d), jnp.float32),
            pltpu.VMEM((EB, d), jnp.float32),
            pltpu.SemaphoreType.DMA,
        ],
    )
    def seg_sum(h_hbm, src_hbm, dst_hbm, zero_hbm, out_hbm,
                acc, src0, src1, dst0, dst1, rows0, rows1, gsem):
        c = lax.axis_index("c")
        s = lax.axis_index("s")
        wid = c * NS + s
        base = wid * per_w

        pltpu.sync_copy(zero_hbm, acc.at[pl.ds(s * zps, zps)])
        plsc.subcore_barrier()

        def load_and_gather(jj, src_s, dst_s, rows_s):
            pltpu.sync_copy(src_hbm.at[pl.ds(base + jj * EB, EB)], src_s)
            pltpu.sync_copy(dst_hbm.at[pl.ds(base + jj * EB, EB)], dst_s)
            pltpu.async_copy(h_hbm.at[src_s], rows_s, gsem)

        slots = ((src0, dst0, rows0), (src1, dst1, rows1))
        load_and_gather(0, *slots[0])
        load_and_gather(1, *slots[1])

        def body(i, carry):
            for k, (src_s, dst_s, rows_s) in enumerate(slots):
                jj = i * 2 + k
                pltpu.make_async_copy(h_hbm.at[src_s], rows_s, gsem).wait()
                pltpu.sync_copy(rows_s, acc.at[dst_s], add=True)

                @pl.when(jj + 2 < nb)
                def _(jj=jj, src_s=src_s, dst_s=dst_s, rows_s=rows_s):
                    load_and_gather(jj + 2, src_s, dst_s, rows_s)
            return carry

        lax.fori_loop(0, nb // 2, body, 0)
        plsc.subcore_barrier()

        pltpu.sync_copy(acc.at[pl.ds(s * zps, zps)],
                        out_hbm.at[c].at[pl.ds(s * zps, zps)])

    return seg_sum


def _linear_relu(parts, w, b, n, d, blk, nc):
    nbk = n // blk

    def body(*refs):
        p_refs, (w_ref, b_ref, o_ref) = refs[:nc], refs[nc:]
        msgs = p_refs[0][0]
        for pr in p_refs[1:]:
            msgs = msgs + pr[0]
        y = lax.dot_general(msgs, w_ref[...], (((1,), (0,)), ((), ())),
                            preferred_element_type=jnp.float32)
        o_ref[...] = jnp.maximum(y + b_ref[...], 0.0)

    in_specs = [
        pl.BlockSpec((1, blk, d), functools.partial(lambda cc, i: (cc, i, 0), cc))
        for cc in range(nc)
    ] + [
        pl.BlockSpec((d, d), lambda i: (0, 0)),
        pl.BlockSpec((1, d), lambda i: (0, 0)),
    ]
    return pl.pallas_call(
        body,
        grid=(nbk,),
        in_specs=in_specs,
        out_specs=pl.BlockSpec((blk, d), lambda i: (i, 0)),
        out_shape=jax.ShapeDtypeStruct((n, d), jnp.float32),
    )(*([parts] * nc), w, b.reshape(1, d))


def kernel(x, edge_index, W1, b1, W2, b2):
    n, d = x.shape
    e = edge_index.shape[1]

    loop = jnp.arange(n, dtype=jnp.int32)
    src = jnp.concatenate([edge_index[0].astype(jnp.int32), loop])
    dst = jnp.concatenate([edge_index[1].astype(jnp.int32), loop])

    nc = 1
    etot = e + n
    nw = nc * NS
    nb = -(-etot // (nw * EB))
    nb += nb % 2
    epad = nw * nb * EB - etot
    src = jnp.concatenate([src, jnp.zeros((epad,), jnp.int32)])
    dst = jnp.concatenate([dst, jnp.full((epad,), n, jnp.int32)])

    nacc = -(-(n + 1) // (8 * NS)) * (8 * NS)
    zeros = jnp.zeros((nacc // NS, d), jnp.float32)

    seg = _make_segment_sum(n, d, nacc, nb, nc)

    parts1 = seg(x, src, dst, zeros)
    h1 = _linear_relu(parts1, W1, b1, n, d, 1000, nc)
    parts2 = seg(h1, src, dst, zeros)
    h2 = _linear_relu(parts2, W2, b2, n, d, 1000, nc)
    return h2
